# baseline (speedup 1.0000x reference)
import jax
import jax.numpy as jnp
from jax import lax
from jax.experimental import pallas as pl
from jax.experimental.pallas import tpu as pltpu
from jax.experimental.pallas import tpu_sc as plsc

NUM_EMB = 1000000
DIM = 32

NC = 2
NS = 16
NW = NC * NS

B_TOTAL = 4096 * 50
B_PER_W = B_TOTAL // NW
CHUNK = 128
N_CHUNK = B_PER_W // CHUNK
GROUP = 10
N_GROUP = N_CHUNK // GROUP


def _body(x_hbm, w_hbm, out_hbm, idx_v, rows_v, sem):
    c = lax.axis_index("c")
    s = lax.axis_index("s")
    wid = s * NC + c

    pltpu.sync_copy(x_hbm.at[wid], idx_v)

    for g in range(N_GROUP):
        cps = []
        for j in range(GROUP):
            cp = pltpu.async_copy(
                w_hbm.at[idx_v.at[g * GROUP + j]],
                rows_v.at[pl.ds(j * CHUNK, CHUNK)],
                sem,
            )
            cps.append(cp)
        for cp in cps:
            cp.wait()
        pltpu.sync_copy(
            rows_v,
            out_hbm.at[pl.ds(wid * B_PER_W + g * GROUP * CHUNK, GROUP * CHUNK)],
        )


TR_BLK = 2048


def _tr_body(wt_ref, out_ref):
    out_ref[...] = wt_ref[...].T


def _transpose_table(wt):
    grid = (NUM_EMB + TR_BLK - 1) // TR_BLK
    return pl.pallas_call(
        _tr_body,
        grid=(grid,),
        in_specs=[pl.BlockSpec((DIM, TR_BLK), lambda i: (0, i))],
        out_specs=pl.BlockSpec((TR_BLK, DIM), lambda i: (i, 0)),
        out_shape=jax.ShapeDtypeStruct((NUM_EMB, DIM), jnp.float32),
    )(wt)


@jax.jit
def _run(x_r, wt):
    weight = _transpose_table(wt)
    mesh = plsc.VectorSubcoreMesh(core_axis_name="c", subcore_axis_name="s")
    return pl.kernel(
        _body,
        out_type=jax.ShapeDtypeStruct((B_TOTAL, DIM), jnp.float32),
        mesh=mesh,
        compiler_params=pltpu.CompilerParams(use_tc_tiling_on_sc=False),
        scratch_types=[
            pltpu.VMEM((N_CHUNK, CHUNK), jnp.int32),
            pltpu.VMEM((GROUP * CHUNK, DIM), jnp.float32),
            pltpu.SemaphoreType.DMA,
        ],
    )(x_r, weight)


def kernel(x, weight):
    B, S = x.shape
    x_r = x.astype(jnp.int32).reshape(NW, N_CHUNK, CHUNK)
    out = _run(x_r, weight.T)
    return out.reshape(B, S, DIM)


# speedup vs baseline: 1.2241x; 1.2241x over previous
import jax
import jax.numpy as jnp
from jax import lax
from jax.experimental import pallas as pl
from jax.experimental.pallas import tpu as pltpu
from jax.experimental.pallas import tpu_sc as plsc

NUM_EMB = 1000000
DIM = 32

NC = 2
NS = 16
NW = NC * NS

B_TOTAL = 4096 * 50
B_PER_W = B_TOTAL // NW
CHUNK = 128
N_CHUNK = B_PER_W // CHUNK
GROUP = 10
N_GROUP = N_CHUNK // GROUP


def _body(x_hbm, w_hbm, out_hbm, idx_v, rows_v, sem):
    c = lax.axis_index("c")
    s = lax.axis_index("s")
    wid = s * NC + c

    pltpu.sync_copy(x_hbm.at[wid], idx_v)

    for g in range(N_GROUP):
        cps = []
        for j in range(GROUP):
            cp = pltpu.async_copy(
                w_hbm.at[idx_v.at[g * GROUP + j]],
                rows_v.at[pl.ds(j * CHUNK, CHUNK)],
                sem,
            )
            cps.append(cp)
        for cp in cps:
            cp.wait()
        pltpu.sync_copy(
            rows_v,
            out_hbm.at[pl.ds(wid * B_PER_W + g * GROUP * CHUNK, GROUP * CHUNK)],
        )


TR_BLK = 8192


def _tr_body(wt_ref, out_ref):
    out_ref[...] = wt_ref[...].T


def _transpose_table(wt):
    grid = (NUM_EMB + TR_BLK - 1) // TR_BLK
    return pl.pallas_call(
        _tr_body,
        grid=(grid,),
        in_specs=[pl.BlockSpec((DIM, TR_BLK), lambda i: (0, i))],
        out_specs=pl.BlockSpec((TR_BLK, DIM), lambda i: (i, 0)),
        out_shape=jax.ShapeDtypeStruct((NUM_EMB, DIM), jnp.float32),
    )(wt)


@jax.jit
def _run(x_r, wt):
    weight = _transpose_table(wt)
    mesh = plsc.VectorSubcoreMesh(core_axis_name="c", subcore_axis_name="s")
    return pl.kernel(
        _body,
        out_type=jax.ShapeDtypeStruct((B_TOTAL, DIM), jnp.float32),
        mesh=mesh,
        compiler_params=pltpu.CompilerParams(use_tc_tiling_on_sc=False),
        scratch_types=[
            pltpu.VMEM((N_CHUNK, CHUNK), jnp.int32),
            pltpu.VMEM((GROUP * CHUNK, DIM), jnp.float32),
            pltpu.SemaphoreType.DMA,
        ],
    )(x_r, weight)


def kernel(x, weight):
    B, S = x.shape
    x_r = x.astype(jnp.int32).reshape(NW, N_CHUNK, CHUNK)
    out = _run(x_r, weight.T)
    return out.reshape(B, S, DIM)


# speedup vs baseline: 1.2514x; 1.0222x over previous
import jax
import jax.numpy as jnp
from jax import lax
from jax.experimental import pallas as pl
from jax.experimental.pallas import tpu as pltpu
from jax.experimental.pallas import tpu_sc as plsc

NUM_EMB = 1000000
DIM = 32

NC = 2
NS = 16
NW = NC * NS

B_TOTAL = 4096 * 50
B_PER_W = B_TOTAL // NW
CHUNK = 128
N_CHUNK = B_PER_W // CHUNK
GROUP = 10
N_GROUP = N_CHUNK // GROUP


def _body(x_hbm, w_hbm, out_hbm, idx_v, rows_v, sem):
    c = lax.axis_index("c")
    s = lax.axis_index("s")
    wid = s * NC + c

    pltpu.sync_copy(x_hbm.at[wid], idx_v)

    for g in range(N_GROUP):
        cps = []
        for j in range(GROUP):
            cp = pltpu.async_copy(
                w_hbm.at[idx_v.at[g * GROUP + j]],
                rows_v.at[pl.ds(j * CHUNK, CHUNK)],
                sem,
            )
            cps.append(cp)
        for cp in cps:
            cp.wait()
        pltpu.sync_copy(
            rows_v,
            out_hbm.at[pl.ds(wid * B_PER_W + g * GROUP * CHUNK, GROUP * CHUNK)],
        )


TR_BLK = 8192


def _tr_body(wt_ref, out_ref):
    ident = jnp.eye(128, dtype=jnp.float32)
    wt = wt_ref[...]
    for k in range(TR_BLK // 128):
        chunk = wt[:, k * 128:(k + 1) * 128]
        out_ref[k * 128:(k + 1) * 128, :] = jax.lax.dot_general(
            ident, chunk, (((1,), (1,)), ((), ())),
            preferred_element_type=jnp.float32)


def _transpose_table(wt):
    grid = (NUM_EMB + TR_BLK - 1) // TR_BLK
    return pl.pallas_call(
        _tr_body,
        grid=(grid,),
        in_specs=[pl.BlockSpec((DIM, TR_BLK), lambda i: (0, i))],
        out_specs=pl.BlockSpec((TR_BLK, DIM), lambda i: (i, 0)),
        out_shape=jax.ShapeDtypeStruct((NUM_EMB, DIM), jnp.float32),
    )(wt)


@jax.jit
def _run(x_r, wt):
    weight = _transpose_table(wt)
    mesh = plsc.VectorSubcoreMesh(core_axis_name="c", subcore_axis_name="s")
    return pl.kernel(
        _body,
        out_type=jax.ShapeDtypeStruct((B_TOTAL, DIM), jnp.float32),
        mesh=mesh,
        compiler_params=pltpu.CompilerParams(use_tc_tiling_on_sc=False),
        scratch_types=[
            pltpu.VMEM((N_CHUNK, CHUNK), jnp.int32),
            pltpu.VMEM((GROUP * CHUNK, DIM), jnp.float32),
            pltpu.SemaphoreType.DMA,
        ],
    )(x_r, weight)


def kernel(x, weight):
    B, S = x.shape
    x_r = x.astype(jnp.int32).reshape(NW, N_CHUNK, CHUNK)
    out = _run(x_r, weight.T)
    return out.reshape(B, S, DIM)


# speedup vs baseline: 1.3906x; 1.1113x over previous
import jax
import jax.numpy as jnp
from jax import lax
from jax.experimental import pallas as pl
from jax.experimental.pallas import tpu as pltpu
from jax.experimental.pallas import tpu_sc as plsc

NUM_EMB = 1000000
DIM = 32

NC = 2
NS = 16
NW = NC * NS

B = 4096
S = 50
NB = B // 128

TR_BLK = 8192


def _tr_body(wt_ref, out_ref):
    out_ref[...] = wt_ref[...].T


def _transpose_table(wt):
    grid = (NUM_EMB + TR_BLK - 1) // TR_BLK
    return pl.pallas_call(
        _tr_body,
        grid=(grid,),
        in_specs=[pl.BlockSpec((DIM, TR_BLK), lambda i: (0, i))],
        out_specs=pl.BlockSpec((TR_BLK, DIM), lambda i: (i, 0)),
        out_shape=jax.ShapeDtypeStruct((NUM_EMB, DIM), jnp.float32),
    )(wt)


def _transpose_block(rows, trans, row_idx, col_splat):
    for c in range(DIM):
        for l0 in range(8):
            v = plsc.load_gather(rows, [row_idx[l0], col_splat[c]])
            trans[c // 8, c % 8, pl.ds(l0 * 16, 16)] = v


def _gather_body(xt_hbm, w_hbm, out_hbm, idx_v, rows_v, trans_v,
                 sg0, sg1, so0, so1):
    c = lax.axis_index("c")
    s = lax.axis_index("s")
    nb = s * NC + c

    pltpu.sync_copy(xt_hbm.at[:, pl.ds(nb * 128, 128)], idx_v)

    iota = lax.iota(jnp.int32, 16)
    row_idx = [iota + (l0 * 16) for l0 in range(8)]
    col_splat = [jnp.full((16,), cc, jnp.int32) for cc in range(DIM)]
    sgs = (sg0, sg1)
    sos = (so0, so1)

    def gather(j, p):
        return pltpu.async_copy(w_hbm.at[idx_v.at[j]], rows_v.at[p], sgs[p])

    def put(j, p):
        return pltpu.async_copy(trans_v.at[p], out_hbm.at[j, :, nb], sos[p])

    gather(0, 0)
    gather(1, 1)

    def step(i, _):
        for p in range(2):
            j = 2 * i + p
            @pl.when(i > 0)
            def _():
                pltpu.make_async_copy(trans_v.at[p], out_hbm.at[j, :, nb],
                                      sos[p]).wait()
            pltpu.make_async_copy(w_hbm.at[idx_v.at[j]], rows_v.at[p],
                                  sgs[p]).wait()
            _transpose_block(rows_v.at[p], trans_v.at[p], row_idx, col_splat)
            put(j, p)

            @pl.when(j + 2 < S)
            def _():
                gather(j + 2, p)
        return None

    lax.fori_loop(0, S // 2, step, None)
    for p in range(2):
        pltpu.make_async_copy(trans_v.at[p], out_hbm.at[0, :, nb],
                              sos[p]).wait()


@jax.jit
def _run(xt, wt):
    w_lin = _transpose_table(wt)
    mesh = plsc.VectorSubcoreMesh(core_axis_name="c", subcore_axis_name="s")
    o5 = pl.kernel(
        _gather_body,
        out_type=jax.ShapeDtypeStruct((S, 4, NB, 8, 128), jnp.float32),
        mesh=mesh,
        compiler_params=pltpu.CompilerParams(use_tc_tiling_on_sc=False,
                                             needs_layout_passes=False),
        scratch_types=[
            pltpu.VMEM((S, 128), jnp.int32),
            pltpu.VMEM((2, 128, DIM), jnp.float32),
            pltpu.VMEM((2, 4, 8, 128), jnp.float32),
            pltpu.SemaphoreType.DMA,
            pltpu.SemaphoreType.DMA,
            pltpu.SemaphoreType.DMA,
            pltpu.SemaphoreType.DMA,
        ],
    )(xt, w_lin)
    return o5.transpose(2, 4, 0, 1, 3).reshape(B, S, DIM)


def kernel(x, weight):
    return _run(x.astype(jnp.int32).T, weight.T)


# speedup vs baseline: 1.5931x; 1.1456x over previous
import jax
import jax.numpy as jnp
from jax import lax
from jax.experimental import pallas as pl
from jax.experimental.pallas import tpu as pltpu
from jax.experimental.pallas import tpu_sc as plsc

NUM_EMB = 1000000
DIM = 32

NC = 2
NS = 16
NW = NC * NS

B = 4096
S = 50
NB = B // 128

TR_BLK = 8192


def _tr_body(wt_ref, out_ref):
    out_ref[...] = wt_ref[...].T


def _transpose_table(wt):
    grid = (NUM_EMB + TR_BLK - 1) // TR_BLK
    return pl.pallas_call(
        _tr_body,
        grid=(grid,),
        in_specs=[pl.BlockSpec((DIM, TR_BLK), lambda i: (0, i))],
        out_specs=pl.BlockSpec((TR_BLK, DIM), lambda i: (i, 0)),
        out_shape=jax.ShapeDtypeStruct((NUM_EMB, DIM), jnp.float32),
    )(wt)


def _transpose_block(rows, trans, row_idx):
    @plsc.parallel_loop(0, DIM, step=1)
    def _(c):
        col = jnp.full((16,), 0, jnp.int32) + c
        for l0 in range(8):
            v = plsc.load_gather(rows, [row_idx[l0], col])
            trans[c, pl.ds(l0 * 16, 16)] = v


def _gather_body(xt_hbm, w_hbm, out_hbm, idx_v, rows_v, trans_v,
                 sg0, sg1, so0, so1):
    c = lax.axis_index("c")
    s = lax.axis_index("s")
    nb = s * NC + c

    pltpu.sync_copy(xt_hbm.at[:, pl.ds(nb * 128, 128)], idx_v)

    iota = lax.iota(jnp.int32, 16)
    row_idx = [iota + (l0 * 16) for l0 in range(8)]
    sgs = (sg0, sg1)
    sos = (so0, so1)

    def gather(j, p):
        return pltpu.async_copy(w_hbm.at[idx_v.at[j]], rows_v.at[p], sgs[p])

    def put(j, p):
        for a in range(4):
            pltpu.async_copy(trans_v.at[p, pl.ds(a * 8, 8)],
                             out_hbm.at[j, a, nb], sos[p])

    def drain_put(p):
        for a in range(4):
            pltpu.make_async_copy(trans_v.at[p, pl.ds(a * 8, 8)],
                                  out_hbm.at[0, a, nb], sos[p]).wait()

    gather(0, 0)
    gather(1, 1)

    def step(i, _):
        for p in range(2):
            j = 2 * i + p
            @pl.when(i > 0)
            def _():
                drain_put(p)
            pltpu.make_async_copy(w_hbm.at[idx_v.at[j]], rows_v.at[p],
                                  sgs[p]).wait()
            _transpose_block(rows_v.at[p], trans_v.at[p], row_idx)
            put(j, p)

            @pl.when(j + 2 < S)
            def _():
                gather(j + 2, p)
        return None

    lax.fori_loop(0, S // 2, step, None)
    for p in range(2):
        drain_put(p)


@jax.jit
def _run(xt, wt):
    w_lin = _transpose_table(wt)
    mesh = plsc.VectorSubcoreMesh(core_axis_name="c", subcore_axis_name="s")
    o5 = pl.kernel(
        _gather_body,
        out_type=jax.ShapeDtypeStruct((S, 4, NB, 8, 128), jnp.float32),
        mesh=mesh,
        compiler_params=pltpu.CompilerParams(use_tc_tiling_on_sc=False,
                                             needs_layout_passes=False),
        scratch_types=[
            pltpu.VMEM((S, 128), jnp.int32),
            pltpu.VMEM((2, 128, DIM), jnp.float32),
            pltpu.VMEM((2, DIM, 128), jnp.float32),
            pltpu.SemaphoreType.DMA,
            pltpu.SemaphoreType.DMA,
            pltpu.SemaphoreType.DMA,
            pltpu.SemaphoreType.DMA,
        ],
    )(xt, w_lin)
    return o5.transpose(2, 4, 0, 1, 3).reshape(B, S, DIM)


def kernel(x, weight):
    return _run(x.astype(jnp.int32).T, weight.T)


# speedup vs baseline: 1.8071x; 1.1343x over previous
import jax
import jax.numpy as jnp
from jax import lax
from jax.experimental import pallas as pl
from jax.experimental.pallas import tpu as pltpu
from jax.experimental.pallas import tpu_sc as plsc

NUM_EMB = 1000000
DIM = 32

NC = 2
NS = 16
NW = NC * NS

B = 4096
S = 50
NB = B // 128

TR_BLK = 8192


def _tr_body(wt_ref, out_ref):
    t1 = wt_ref[...].T
    sel = jax.lax.broadcasted_iota(jnp.int32, (TR_BLK // 4, DIM), 0) * 4
    for j in range(4):
        out_ref[:, 32 * j:32 * (j + 1)] = jnp.take_along_axis(
            t1, sel + j, axis=0)


def _transpose_table(wt):
    grid = (NUM_EMB + TR_BLK - 1) // TR_BLK
    packed = pl.pallas_call(
        _tr_body,
        grid=(grid,),
        in_specs=[pl.BlockSpec((DIM, TR_BLK), lambda i: (0, i))],
        out_specs=pl.BlockSpec((TR_BLK // 4, 128), lambda i: (i, 0)),
        out_shape=jax.ShapeDtypeStruct((NUM_EMB // 4, 128), jnp.float32),
    )(wt)
    return packed.reshape(NUM_EMB, DIM)


def _transpose_block(rows, trans, row_idx):
    @plsc.parallel_loop(0, DIM, step=1)
    def _(c):
        col = jnp.full((16,), 0, jnp.int32) + c
        for l0 in range(8):
            v = plsc.load_gather(rows, [row_idx[l0], col])
            trans[c, pl.ds(l0 * 16, 16)] = v


def _gather_body(xt_hbm, w_hbm, out_hbm, idx_v, rows_v, trans_v,
                 sg0, sg1, so0, so1):
    c = lax.axis_index("c")
    s = lax.axis_index("s")
    nb = s * NC + c

    pltpu.sync_copy(xt_hbm.at[:, pl.ds(nb * 128, 128)], idx_v)

    iota = lax.iota(jnp.int32, 16)
    row_idx = [iota + (l0 * 16) for l0 in range(8)]
    sgs = (sg0, sg1)
    sos = (so0, so1)

    def gather(j, p):
        return pltpu.async_copy(w_hbm.at[idx_v.at[j]], rows_v.at[p], sgs[p])

    def put(j, p):
        for a in range(4):
            pltpu.async_copy(trans_v.at[p, pl.ds(a * 8, 8)],
                             out_hbm.at[j, a, nb], sos[p])

    def drain_put(p):
        for a in range(4):
            pltpu.make_async_copy(trans_v.at[p, pl.ds(a * 8, 8)],
                                  out_hbm.at[0, a, nb], sos[p]).wait()

    gather(0, 0)
    gather(1, 1)

    def step(i, _):
        for p in range(2):
            j = 2 * i + p
            @pl.when(i > 0)
            def _():
                drain_put(p)
            pltpu.make_async_copy(w_hbm.at[idx_v.at[j]], rows_v.at[p],
                                  sgs[p]).wait()
            _transpose_block(rows_v.at[p], trans_v.at[p], row_idx)
            put(j, p)

            @pl.when(j + 2 < S)
            def _():
                gather(j + 2, p)
        return None

    lax.fori_loop(0, S // 2, step, None)
    for p in range(2):
        drain_put(p)


@jax.jit
def _run(xt, w_lin):
    mesh = plsc.VectorSubcoreMesh(core_axis_name="c", subcore_axis_name="s")
    o5 = pl.kernel(
        _gather_body,
        out_type=jax.ShapeDtypeStruct((S, 4, NB, 8, 128), jnp.float32),
        mesh=mesh,
        compiler_params=pltpu.CompilerParams(use_tc_tiling_on_sc=False,
                                             needs_layout_passes=False),
        scratch_types=[
            pltpu.VMEM((S, 128), jnp.int32),
            pltpu.VMEM((2, 128, DIM), jnp.float32),
            pltpu.VMEM((2, DIM, 128), jnp.float32),
            pltpu.SemaphoreType.DMA,
            pltpu.SemaphoreType.DMA,
            pltpu.SemaphoreType.DMA,
            pltpu.SemaphoreType.DMA,
        ],
    )(xt, w_lin)
    return o5.transpose(2, 4, 0, 1, 3).reshape(B, S, DIM)


def kernel(x, weight):
    return _run(x.astype(jnp.int32).T, weight)


# speedup vs baseline: 3.1312x; 1.7327x over previous
import jax
import jax.numpy as jnp
from jax import lax
from jax.experimental import pallas as pl
from jax.experimental.pallas import tpu as pltpu
from jax.experimental.pallas import tpu_sc as plsc

NUM_EMB = 1000000
DIM = 32

NC = 2
NS = 16
NW = NC * NS

B = 4096
S = 50
NB = B // 128

TR_BLK = 8192


def _tr_body(wt_ref, out_ref):
    out_ref[:, 0:DIM] = wt_ref[...].T


def _transpose_table(wt):
    grid = (NUM_EMB + TR_BLK - 1) // TR_BLK
    return pl.pallas_call(
        _tr_body,
        grid=(grid,),
        in_specs=[pl.BlockSpec((DIM, TR_BLK), lambda i: (0, i))],
        out_specs=pl.BlockSpec((TR_BLK, 128), lambda i: (i, 0)),
        out_shape=jax.ShapeDtypeStruct((NUM_EMB, 128), jnp.float32),
    )(wt)


def _transpose_block(rows, trans, row_idx):
    @plsc.parallel_loop(0, DIM, step=1)
    def _(c):
        col = jnp.full((16,), 0, jnp.int32) + c
        for l0 in range(8):
            v = plsc.load_gather(rows, [row_idx[l0], col])
            trans[c, pl.ds(l0 * 16, 16)] = v


def _gather_body(xt_hbm, w_hbm, out_hbm, idx_v, rows_v, trans_v,
                 sg0, sg1, so0, so1):
    c = lax.axis_index("c")
    s = lax.axis_index("s")
    nb = s * NC + c

    pltpu.sync_copy(xt_hbm.at[:, pl.ds(nb * 128, 128)], idx_v)

    iota = lax.iota(jnp.int32, 16)
    row_idx = [iota + (l0 * 16) for l0 in range(8)]
    sgs = (sg0, sg1)
    sos = (so0, so1)

    def gather(j, p):
        return pltpu.async_copy(w_hbm.at[idx_v.at[j]], rows_v.at[p], sgs[p])

    def put(j, p):
        for a in range(4):
            pltpu.async_copy(trans_v.at[p, pl.ds(a * 8, 8)],
                             out_hbm.at[j, a, nb], sos[p])

    def drain_put(p):
        for a in range(4):
            pltpu.make_async_copy(trans_v.at[p, pl.ds(a * 8, 8)],
                                  out_hbm.at[0, a, nb], sos[p]).wait()

    gather(0, 0)
    gather(1, 1)

    def step(i, _):
        for p in range(2):
            j = 2 * i + p
            @pl.when(i > 0)
            def _():
                drain_put(p)
            pltpu.make_async_copy(w_hbm.at[idx_v.at[j]], rows_v.at[p],
                                  sgs[p]).wait()
            _transpose_block(rows_v.at[p], trans_v.at[p], row_idx)
            put(j, p)

            @pl.when(j + 2 < S)
            def _():
                gather(j + 2, p)
        return None

    lax.fori_loop(0, S // 2, step, None)
    for p in range(2):
        drain_put(p)


@jax.jit
def _run(xt, wt):
    w_lin = _transpose_table(wt)
    mesh = plsc.VectorSubcoreMesh(core_axis_name="c", subcore_axis_name="s")
    o5 = pl.kernel(
        _gather_body,
        out_type=jax.ShapeDtypeStruct((S, 4, NB, 8, 128), jnp.float32),
        mesh=mesh,
        compiler_params=pltpu.CompilerParams(use_tc_tiling_on_sc=False,
                                             needs_layout_passes=False),
        scratch_types=[
            pltpu.VMEM((S, 128), jnp.int32),
            pltpu.VMEM((2, 128, 128), jnp.float32),
            pltpu.VMEM((2, DIM, 128), jnp.float32),
            pltpu.SemaphoreType.DMA,
            pltpu.SemaphoreType.DMA,
            pltpu.SemaphoreType.DMA,
            pltpu.SemaphoreType.DMA,
        ],
    )(xt, w_lin)
    return o5.transpose(2, 4, 0, 1, 3).reshape(B, S, DIM)


def kernel(x, weight):
    return _run(x.astype(jnp.int32).T, weight.T)


# speedup vs baseline: 3.4604x; 1.1051x over previous
import jax
import jax.numpy as jnp
from jax import lax
from jax.experimental import pallas as pl
from jax.experimental.pallas import tpu as pltpu
from jax.experimental.pallas import tpu_sc as plsc

NUM_EMB = 1000000
DIM = 32

NC = 2
NS = 16
NW = NC * NS

B = 4096
S = 50
NB = B // 128

TR_BLK = 16384


def _tr_body(wt_ref, out_ref):
    out_ref[:, 0:DIM] = wt_ref[...].T


def _transpose_table(wt):
    grid = (NUM_EMB + TR_BLK - 1) // TR_BLK
    return pl.pallas_call(
        _tr_body,
        grid=(grid,),
        in_specs=[pl.BlockSpec((DIM, TR_BLK), lambda i: (0, i))],
        out_specs=pl.BlockSpec((TR_BLK, 128), lambda i: (i, 0)),
        out_shape=jax.ShapeDtypeStruct((NUM_EMB, 128), jnp.float32),
    )(wt)


def _transpose_block(rows, trans, row_idx):
    @plsc.parallel_loop(0, DIM, step=1, unroll=4)
    def _(c):
        col = jnp.full((16,), 0, jnp.int32) + c
        for l0 in range(8):
            v = plsc.load_gather(rows, [row_idx[l0], col])
            trans[c, pl.ds(l0 * 16, 16)] = v


def _gather_body(xt_hbm, w_hbm, out_hbm, idx_v, rows_v, trans_v,
                 sg0, sg1, so0, so1):
    c = lax.axis_index("c")
    s = lax.axis_index("s")
    nb = s * NC + c

    pltpu.sync_copy(xt_hbm.at[:, pl.ds(nb * 128, 128)], idx_v)

    iota = lax.iota(jnp.int32, 16)
    row_idx = [iota + (l0 * 16) for l0 in range(8)]
    sgs = (sg0, sg1)
    sos = (so0, so1)

    def gather(j, p):
        return pltpu.async_copy(w_hbm.at[idx_v.at[j]], rows_v.at[p], sgs[p])

    def put(j, p):
        for a in range(4):
            pltpu.async_copy(trans_v.at[p, pl.ds(a * 8, 8)],
                             out_hbm.at[j, a, nb], sos[p])

    def drain_put(p):
        for a in range(4):
            pltpu.make_async_copy(trans_v.at[p, pl.ds(a * 8, 8)],
                                  out_hbm.at[0, a, nb], sos[p]).wait()

    gather(0, 0)
    gather(1, 1)

    def step(i, _):
        for p in range(2):
            j = 2 * i + p
            @pl.when(i > 0)
            def _():
                drain_put(p)
            pltpu.make_async_copy(w_hbm.at[idx_v.at[j]], rows_v.at[p],
                                  sgs[p]).wait()
            _transpose_block(rows_v.at[p], trans_v.at[p], row_idx)
            put(j, p)

            @pl.when(j + 2 < S)
            def _():
                gather(j + 2, p)
        return None

    lax.fori_loop(0, S // 2, step, None)
    for p in range(2):
        drain_put(p)


@jax.jit
def _run(xt, wt):
    w_lin = _transpose_table(wt)
    mesh = plsc.VectorSubcoreMesh(core_axis_name="c", subcore_axis_name="s")
    o5 = pl.kernel(
        _gather_body,
        out_type=jax.ShapeDtypeStruct((S, 4, NB, 8, 128), jnp.float32),
        mesh=mesh,
        compiler_params=pltpu.CompilerParams(use_tc_tiling_on_sc=False,
                                             needs_layout_passes=False),
        scratch_types=[
            pltpu.VMEM((S, 128), jnp.int32),
            pltpu.VMEM((2, 128, 128), jnp.float32),
            pltpu.VMEM((2, DIM, 128), jnp.float32),
            pltpu.SemaphoreType.DMA,
            pltpu.SemaphoreType.DMA,
            pltpu.SemaphoreType.DMA,
            pltpu.SemaphoreType.DMA,
        ],
    )(xt, w_lin)
    return o5.transpose(2, 4, 0, 1, 3).reshape(B, S, DIM)


def kernel(x, weight):
    return _run(x.astype(jnp.int32).T, weight.T)
